# baseline (speedup 1.0000x reference)
import jax
import jax.numpy as jnp
from jax import lax
from jax.experimental import pallas as pl
from jax.experimental.pallas import tpu as pltpu
from jax.experimental.pallas import tpu_sc as plsc

_NC, _NS, _L = 2, 16, 16
_NW = _NC * _NS
_TOTAL = 16384 * 200
_NPER = _TOTAL // _NW
_CHUNK = 12800
_NCHUNK = _NPER // _CHUNK
_TBL = 128


def _sc_body(conv_hbm, sp_hbm, out_hbm, conv_v, in0, in1, out0, out1,
             si0, si1, so0, so1):
    c = lax.axis_index("c")
    s = lax.axis_index("s")
    base = (s * _NC + c) * _NPER
    pltpu.sync_copy(conv_hbm, conv_v)
    ins, outs = (in0, in1), (out0, out1)
    isems, osems = (si0, si1), (so0, so1)
    in_cp = [None, None]
    out_cp = [None, None]
    in_cp[0] = pltpu.async_copy(sp_hbm.at[pl.ds(base, _CHUNK)], ins[0], isems[0])
    for g in range(_NCHUNK):
        b = g & 1
        nb = b ^ 1
        if g + 1 < _NCHUNK:
            in_cp[nb] = pltpu.async_copy(
                sp_hbm.at[pl.ds(base + (g + 1) * _CHUNK, _CHUNK)], ins[nb], isems[nb])
        in_cp[b].wait()
        if out_cp[b] is not None:
            out_cp[b].wait()

        @plsc.parallel_loop(0, _CHUNK, step=_L, unroll=8)
        def _(i, _ib=ins[b], _ob=outs[b]):
            _ob[pl.ds(i, _L)] = plsc.load_gather(conv_v, [_ib[pl.ds(i, _L)]])

        out_cp[b] = pltpu.async_copy(outs[b], out_hbm.at[pl.ds(base + g * _CHUNK, _CHUNK)], osems[b])
    for b in range(2):
        if out_cp[b] is not None:
            out_cp[b].wait()


def kernel(species, coordinates, conv_tensor):
    sp = species.reshape(_TOTAL)
    conv = jnp.zeros((_TBL,), conv_tensor.dtype).at[:conv_tensor.shape[0]].set(conv_tensor)
    lookup = pl.kernel(
        _sc_body,
        out_type=jax.ShapeDtypeStruct((_TOTAL,), sp.dtype),
        mesh=plsc.VectorSubcoreMesh(
            core_axis_name="c", subcore_axis_name="s",
            num_cores=_NC, num_subcores=_NS),
        scratch_types=[
            pltpu.VMEM((_TBL,), jnp.int32),
            pltpu.VMEM((_CHUNK,), jnp.int32),
            pltpu.VMEM((_CHUNK,), jnp.int32),
            pltpu.VMEM((_CHUNK,), jnp.int32),
            pltpu.VMEM((_CHUNK,), jnp.int32),
            pltpu.SemaphoreType.DMA,
            pltpu.SemaphoreType.DMA,
            pltpu.SemaphoreType.DMA,
            pltpu.SemaphoreType.DMA,
        ],
        compiler_params=pltpu.CompilerParams(needs_layout_passes=False),
    )
    out = lookup(conv, sp)
    return out.reshape(species.shape), coordinates
